# Initial kernel scaffold; baseline (speedup 1.0000x reference)
import jax
import jax.numpy as jnp
from jax.experimental import pallas as pl


def kernel(values, logits):
    raise NotImplementedError("write your pallas kernel here")


# speedup vs baseline: 279.9706x; 279.9706x over previous
import functools

import jax
import jax.numpy as jnp
from jax.experimental import pallas as pl
from jax.experimental.pallas import tpu as pltpu

_TEMPERATURE = 0.5
_TD = 512


def _routing_idx(values, logits):
    key = jax.random.key(42)
    u = jax.random.uniform(
        key, logits.shape, minval=0.0, maxval=1.0, dtype=jnp.float32
    )
    g = -jnp.log(-jnp.log(u + 1e-20) + 1e-20)
    y = jax.nn.softmax((logits + g) / _TEMPERATURE, axis=-1)
    C = logits.shape[2]
    clusters = jnp.arange(C, dtype=jnp.float32)
    soft = jnp.sum(y * clusters, axis=2, keepdims=True)
    return jax.lax.stop_gradient(soft).astype(jnp.int32)[..., 0]


def _tc_body(f_ref, v_ref, o_ref):
    k = pl.program_id(1)
    C = o_ref.shape[1]
    f = f_ref[0, 0]
    iota_c = jax.lax.broadcasted_iota(jnp.int32, (C, 1), 0).astype(jnp.float32)
    ohT = (f == iota_c).astype(jnp.float32)
    acc = jax.lax.dot_general(
        ohT,
        v_ref[0],
        (((1,), (0,)), ((), ())),
        preferred_element_type=jnp.float32,
        precision=jax.lax.Precision.HIGHEST,
    )

    @pl.when(k == 0)
    def _init():
        o_ref[0] = acc

    @pl.when(k != 0)
    def _acc():
        o_ref[0] += acc


def _scatter_tc(fidx, values, C):
    B, D, F = values.shape
    NCH = D // _TD
    fidx_r = fidx.reshape(B, NCH, 1, _TD)
    grid = (B, NCH)
    return pl.pallas_call(
        _tc_body,
        grid=grid,
        in_specs=[
            pl.BlockSpec((1, 1, 1, _TD), lambda b, k: (b, k, 0, 0)),
            pl.BlockSpec((1, _TD, F), lambda b, k: (b, k, 0)),
        ],
        out_specs=pl.BlockSpec((1, C, F), lambda b, k: (b, 0, 0)),
        out_shape=jax.ShapeDtypeStruct((B, C, F), jnp.float32),
    )(fidx_r, values)


def kernel(values, logits):
    C = logits.shape[2]
    idx = _routing_idx(values, logits)
    fidx = idx.astype(jnp.float32)
    return _scatter_tc(fidx, values, C)
